# baseline (speedup 1.0000x reference)
import functools

import jax
import jax.numpy as jnp
from jax import lax
from jax.experimental import pallas as pl
from jax.experimental.pallas import tpu as pltpu
from jax.experimental.pallas import tpu_sc as plsc

N_NODES = 10000
N_PAD = 10240
N_EDGES = 320000
IN_FEATS = 128
HIDDEN = 64

NC = 2
NS = 16
NW = NC * NS
LANES = 128
N_CHUNKS = N_EDGES // LANES
BASE_CNT = N_CHUNKS // NW
REM = N_CHUNKS % NW
MAX_CNT = BASE_CNT + 1
SLOTS = 4
ROWS_PER_TILE = N_PAD // NS
ZROWS = 40

ROW_BLK = 1024
WDEG = 16

_P = jax.lax.Precision.HIGHEST


def _dot(a, b):
    return jax.lax.dot_general(a, b, (((1,), (0,)), ((), ())),
                               preferred_element_type=jnp.float32,
                               precision=_P)


def _stage1_body(feats_ref, wcat_ref, y_ref, z_ref):
    acc = _dot(feats_ref[...], wcat_ref[...])
    y_ref[...] = acc[:, :HIDDEN]
    z_ref[...] = acc[:, HIDDEN:]


def _stage1(feats, wcat):
    return pl.pallas_call(
        _stage1_body,
        grid=(N_PAD // ROW_BLK,),
        in_specs=[
            pl.BlockSpec((ROW_BLK, IN_FEATS), lambda i: (i, 0)),
            pl.BlockSpec((IN_FEATS, 2 * HIDDEN), lambda i: (0, 0)),
        ],
        out_specs=[
            pl.BlockSpec((ROW_BLK, HIDDEN), lambda i: (i, 0)),
            pl.BlockSpec((ROW_BLK, HIDDEN), lambda i: (i, 0)),
        ],
        out_shape=[
            jax.ShapeDtypeStruct((N_PAD, HIDDEN), jnp.float32),
            jax.ShapeDtypeStruct((N_PAD, HIDDEN), jnp.float32),
        ],
    )(feats, wcat)


def _stage2_body(z1_ref, agg_ref, degp_ref, b1_ref, wn2_ref, ws2_ref,
                 y2_ref, z2_ref, deg_ref):
    a = agg_ref[0] + agg_ref[1]
    deg = jnp.maximum(degp_ref[0, :, 0:1] + degp_ref[1, :, 0:1], 1.0)
    mean = a / deg
    h1 = jnp.maximum(z1_ref[...] + mean + b1_ref[...], 0.0)
    y2_ref[...] = _dot(h1, wn2_ref[...])
    z2_ref[...] = _dot(h1, ws2_ref[...])
    deg_ref[...] = jnp.broadcast_to(deg, (ROW_BLK, 8))


def _stage2(z1, agg1, degp, b1, wn2, ws2):
    return pl.pallas_call(
        _stage2_body,
        grid=(N_PAD // ROW_BLK,),
        in_specs=[
            pl.BlockSpec((ROW_BLK, HIDDEN), lambda i: (i, 0)),
            pl.BlockSpec((NC, ROW_BLK, HIDDEN), lambda i: (0, i, 0)),
            pl.BlockSpec((NC, ROW_BLK, WDEG), lambda i: (0, i, 0)),
            pl.BlockSpec((1, HIDDEN), lambda i: (0, 0)),
            pl.BlockSpec((HIDDEN, HIDDEN), lambda i: (0, 0)),
            pl.BlockSpec((HIDDEN, HIDDEN), lambda i: (0, 0)),
        ],
        out_specs=[
            pl.BlockSpec((ROW_BLK, HIDDEN), lambda i: (i, 0)),
            pl.BlockSpec((ROW_BLK, HIDDEN), lambda i: (i, 0)),
            pl.BlockSpec((ROW_BLK, 8), lambda i: (i, 0)),
        ],
        out_shape=[
            jax.ShapeDtypeStruct((N_PAD, HIDDEN), jnp.float32),
            jax.ShapeDtypeStruct((N_PAD, HIDDEN), jnp.float32),
            jax.ShapeDtypeStruct((N_PAD, 8), jnp.float32),
        ],
    )(z1, agg1, degp, b1, wn2, ws2)


def _stage3_body(z2_ref, agg_ref, deg_ref, b2_ref, wout_ref, bout_ref, o_ref):
    a = agg_ref[0] + agg_ref[1]
    mean = a / deg_ref[:, 0:1]
    h2 = jnp.maximum(z2_ref[...] + mean + b2_ref[...], 0.0)
    o_ref[...] = _dot(h2, wout_ref[...]) + bout_ref[...]


def _stage3(z2, agg2, deg, b2, wout_p, bout_p):
    return pl.pallas_call(
        _stage3_body,
        grid=(N_PAD // ROW_BLK,),
        in_specs=[
            pl.BlockSpec((ROW_BLK, HIDDEN), lambda i: (i, 0)),
            pl.BlockSpec((NC, ROW_BLK, HIDDEN), lambda i: (0, i, 0)),
            pl.BlockSpec((ROW_BLK, 8), lambda i: (i, 0)),
            pl.BlockSpec((1, HIDDEN), lambda i: (0, 0)),
            pl.BlockSpec((HIDDEN, 8), lambda i: (0, 0)),
            pl.BlockSpec((1, 8), lambda i: (0, 0)),
        ],
        out_specs=pl.BlockSpec((ROW_BLK, 8), lambda i: (i, 0)),
        out_shape=jax.ShapeDtypeStruct((N_PAD, 8), jnp.float32),
    )(z2, agg2, deg, b2, wout_p, bout_p)


def _make_scatter(width, with_deg, slots):
    mesh = plsc.VectorSubcoreMesh(core_axis_name="c", subcore_axis_name="s")

    out_type = [jax.ShapeDtypeStruct((NC, N_PAD, width), jnp.float32)]
    scratch = [
        pltpu.VMEM((slots, LANES), jnp.int32),
        pltpu.VMEM((slots, LANES), jnp.int32),
        pltpu.VMEM((slots * LANES, width), jnp.float32),
        pltpu.VMEM((ZROWS, width), jnp.float32),
        pltpu.VMEM_SHARED((N_PAD, width), jnp.float32),
        [pltpu.SemaphoreType.DMA] * slots,
        [pltpu.SemaphoreType.DMA] * slots,
    ]
    if with_deg:
        out_type.append(jax.ShapeDtypeStruct((NC, N_PAD, WDEG), jnp.float32))
        scratch += [
            pltpu.VMEM((LANES, WDEG), jnp.float32),
            pltpu.VMEM((ZROWS, WDEG), jnp.float32),
            pltpu.VMEM_SHARED((N_PAD, WDEG), jnp.float32),
            [pltpu.SemaphoreType.DMA] * slots,
        ]

    @functools.partial(
        pl.kernel,
        out_type=out_type,
        mesh=mesh,
        scratch_types=scratch,
        compiler_params=pltpu.CompilerParams(use_tc_tiling_on_sc=False),
    )
    def scat(y_hbm, src_hbm, dst_hbm, out_hbm, *rest):
        if with_deg:
            (outd_hbm, sbuf, dbuf, rows, zbuf, agg_sh, semg, sems,
             ones, zbufd, deg_sh, semd) = rest
        else:
            sbuf, dbuf, rows, zbuf, agg_sh, semg, sems = rest
        cid = lax.axis_index("c")
        sid = lax.axis_index("s")
        gwid = sid * NC + cid
        nlanes = width // 16

        cnt = BASE_CNT + (gwid < REM).astype(jnp.int32)

        def gather_desc(u):
            return pltpu.make_async_copy(
                y_hbm.at[sbuf.at[u]],
                rows.at[pl.ds(u * LANES, LANES)], semg[u])

        def scatter_desc(u):
            return pltpu.make_async_copy(
                rows.at[pl.ds(u * LANES, LANES)],
                agg_sh.at[dbuf.at[u]], sems[u])

        def deg_desc(u):
            return pltpu.make_async_copy(ones, deg_sh.at[dbuf.at[u]], semd[u])

        def slot_wait(u):
            scatter_desc(u).wait()
            if with_deg:
                deg_desc(u).wait()

        def fill(c, u):
            e0 = (gwid + NW * c) * LANES
            pltpu.sync_copy(src_hbm.at[pl.ds(e0, LANES)], sbuf.at[u])
            pltpu.sync_copy(dst_hbm.at[pl.ds(e0, LANES)], dbuf.at[u])
            pltpu.async_copy(y_hbm.at[sbuf.at[u]],
                             rows.at[pl.ds(u * LANES, LANES)], semg[u])

        for u in range(slots - 1):
            fill(u, u)

        def zrow(i, carry):
            for l in range(nlanes):
                zbuf[i, pl.ds(l * 16, 16)] = jnp.zeros((16,), jnp.float32)
            if with_deg:
                zbufd[i, pl.ds(0, 16)] = jnp.zeros((16,), jnp.float32)
            return carry

        lax.fori_loop(0, ZROWS, zrow, 0)
        if with_deg:
            def onesrow(i, carry):
                ones[i, pl.ds(0, 16)] = jnp.ones((16,), jnp.float32)
                return carry

            lax.fori_loop(0, LANES, onesrow, 0)

        def zcopy(k, carry):
            r0 = sid * ROWS_PER_TILE + k * ZROWS
            pltpu.sync_copy(zbuf, agg_sh.at[pl.ds(r0, ZROWS)])
            if with_deg:
                pltpu.sync_copy(zbufd, deg_sh.at[pl.ds(r0, ZROWS)])
            return carry

        lax.fori_loop(0, ROWS_PER_TILE // ZROWS, zcopy, 0)
        plsc.subcore_barrier()

        def body(k, carry):
            for u in range(slots):
                c = slots * k + u

                @pl.when(c < cnt)
                def _():
                    gather_desc(u).wait()
                    pltpu.async_copy(rows.at[pl.ds(u * LANES, LANES)],
                                     agg_sh.at[dbuf.at[u]], sems[u], add=True)
                    if with_deg:
                        pltpu.async_copy(ones, deg_sh.at[dbuf.at[u]],
                                         semd[u], add=True)

                w = (u + slots - 1) % slots

                @pl.when(c + slots - 1 < cnt)
                def _():
                    @pl.when(c >= 1)
                    def _():
                        slot_wait(w)

                    fill(c + slots - 1, w)

            return carry

        lax.fori_loop(0, (MAX_CNT + slots - 1) // slots, body, 0)
        for u in range(slots):
            slot_wait(u)
        plsc.subcore_barrier()
        r0 = sid * ROWS_PER_TILE
        pltpu.sync_copy(agg_sh.at[pl.ds(r0, ROWS_PER_TILE)],
                        out_hbm.at[cid, pl.ds(r0, ROWS_PER_TILE)])
        if with_deg:
            pltpu.sync_copy(deg_sh.at[pl.ds(r0, ROWS_PER_TILE)],
                            outd_hbm.at[cid, pl.ds(r0, ROWS_PER_TILE)])

    return scat


_scatter_deg = _make_scatter(HIDDEN, True, 2)
_scatter_h = _make_scatter(HIDDEN, False, 4)


def kernel(feats, edge_index, W_self1, W_neigh1, b1, W_self2, W_neigh2, b2,
           W_out, b_out):
    src = edge_index[0].astype(jnp.int32)
    dst = edge_index[1].astype(jnp.int32)
    wcat1 = jnp.concatenate([W_neigh1, W_self1], axis=1)

    y1, z1 = _stage1(feats, wcat1)
    agg1, degp = _scatter_deg(y1, src, dst)
    y2, z2, deg = _stage2(z1, agg1, degp, b1.reshape(1, HIDDEN),
                          W_neigh2, W_self2)
    agg2, = _scatter_h(y2, src, dst)

    wout_p = jnp.pad(W_out, ((0, 0), (0, 8 - W_out.shape[1])))
    bout_p = jnp.pad(b_out, (0, 8 - b_out.shape[0])).reshape(1, 8)
    out = _stage3(z2, agg2, deg, b2.reshape(1, HIDDEN), wout_p, bout_p)
    return out[:N_NODES, :W_out.shape[1]]


# speedup vs baseline: 1.1496x; 1.1496x over previous
import functools

import jax
import jax.numpy as jnp
from jax import lax
from jax.experimental import pallas as pl
from jax.experimental.pallas import tpu as pltpu
from jax.experimental.pallas import tpu_sc as plsc

N_NODES = 10000
N_PAD = 10240
N_EDGES = 320000
IN_FEATS = 128
HIDDEN = 64

NC = 2
NS = 16
NW = NC * NS
LANES = 128
R_SC = 2
CHUNK = R_SC * LANES
N_CHUNKS = N_EDGES // CHUNK
BASE_CNT = N_CHUNKS // NW
REM = N_CHUNKS % NW
MAX_CNT = BASE_CNT + 1
ROWS_PER_TILE = N_PAD // NS
ZROWS = 40

ROW_BLK = 1024
W1 = HIDDEN + 16

_P = jax.lax.Precision.HIGHEST


def _dot(a, b):
    return jax.lax.dot_general(a, b, (((1,), (0,)), ((), ())),
                               preferred_element_type=jnp.float32,
                               precision=_P)


def _stage1_body(feats_ref, wcat_ref, y_ref, z_ref):
    acc = _dot(feats_ref[...], wcat_ref[...])
    ones = jnp.ones((ROW_BLK, W1 - HIDDEN), jnp.float32)
    y_ref[...] = jnp.concatenate([acc[:, :HIDDEN], ones], axis=1)
    z_ref[...] = acc[:, HIDDEN:]


def _stage1(feats, wcat):
    return pl.pallas_call(
        _stage1_body,
        grid=(N_PAD // ROW_BLK,),
        in_specs=[
            pl.BlockSpec((ROW_BLK, IN_FEATS), lambda i: (i, 0)),
            pl.BlockSpec((IN_FEATS, 2 * HIDDEN), lambda i: (0, 0)),
        ],
        out_specs=[
            pl.BlockSpec((ROW_BLK, W1), lambda i: (i, 0)),
            pl.BlockSpec((ROW_BLK, HIDDEN), lambda i: (i, 0)),
        ],
        out_shape=[
            jax.ShapeDtypeStruct((N_PAD, W1), jnp.float32),
            jax.ShapeDtypeStruct((N_PAD, HIDDEN), jnp.float32),
        ],
    )(feats, wcat)


def _stage2_body(z1_ref, agg_ref, b1_ref, wn2_ref, ws2_ref,
                 y2_ref, z2_ref, deg_ref):
    a = agg_ref[0] + agg_ref[1]
    deg = jnp.maximum(a[:, HIDDEN:HIDDEN + 1], 1.0)
    mean = a[:, :HIDDEN] / deg
    h1 = jnp.maximum(z1_ref[...] + mean + b1_ref[...], 0.0)
    y2_ref[...] = _dot(h1, wn2_ref[...])
    z2_ref[...] = _dot(h1, ws2_ref[...])
    deg_ref[...] = jnp.broadcast_to(deg, (ROW_BLK, 8))


def _stage2(z1, agg1, b1, wn2, ws2):
    return pl.pallas_call(
        _stage2_body,
        grid=(N_PAD // ROW_BLK,),
        in_specs=[
            pl.BlockSpec((ROW_BLK, HIDDEN), lambda i: (i, 0)),
            pl.BlockSpec((NC, ROW_BLK, W1), lambda i: (0, i, 0)),
            pl.BlockSpec((1, HIDDEN), lambda i: (0, 0)),
            pl.BlockSpec((HIDDEN, HIDDEN), lambda i: (0, 0)),
            pl.BlockSpec((HIDDEN, HIDDEN), lambda i: (0, 0)),
        ],
        out_specs=[
            pl.BlockSpec((ROW_BLK, HIDDEN), lambda i: (i, 0)),
            pl.BlockSpec((ROW_BLK, HIDDEN), lambda i: (i, 0)),
            pl.BlockSpec((ROW_BLK, 8), lambda i: (i, 0)),
        ],
        out_shape=[
            jax.ShapeDtypeStruct((N_PAD, HIDDEN), jnp.float32),
            jax.ShapeDtypeStruct((N_PAD, HIDDEN), jnp.float32),
            jax.ShapeDtypeStruct((N_PAD, 8), jnp.float32),
        ],
    )(z1, agg1, b1, wn2, ws2)


def _stage3_body(z2_ref, agg_ref, deg_ref, b2_ref, wout_ref, bout_ref, o_ref):
    a = agg_ref[0] + agg_ref[1]
    mean = a / deg_ref[:, 0:1]
    h2 = jnp.maximum(z2_ref[...] + mean + b2_ref[...], 0.0)
    o_ref[...] = _dot(h2, wout_ref[...]) + bout_ref[...]


def _stage3(z2, agg2, deg, b2, wout_p, bout_p):
    return pl.pallas_call(
        _stage3_body,
        grid=(N_PAD // ROW_BLK,),
        in_specs=[
            pl.BlockSpec((ROW_BLK, HIDDEN), lambda i: (i, 0)),
            pl.BlockSpec((NC, ROW_BLK, HIDDEN), lambda i: (0, i, 0)),
            pl.BlockSpec((ROW_BLK, 8), lambda i: (i, 0)),
            pl.BlockSpec((1, HIDDEN), lambda i: (0, 0)),
            pl.BlockSpec((HIDDEN, 8), lambda i: (0, 0)),
            pl.BlockSpec((1, 8), lambda i: (0, 0)),
        ],
        out_specs=pl.BlockSpec((ROW_BLK, 8), lambda i: (i, 0)),
        out_shape=jax.ShapeDtypeStruct((N_PAD, 8), jnp.float32),
    )(z2, agg2, deg, b2, wout_p, bout_p)


def _make_scatter(width):
    mesh = plsc.VectorSubcoreMesh(core_axis_name="c", subcore_axis_name="s")

    @functools.partial(
        pl.kernel,
        out_type=jax.ShapeDtypeStruct((NC, N_PAD, width), jnp.float32),
        mesh=mesh,
        scratch_types=[
            pltpu.VMEM((2, CHUNK), jnp.int32),
            pltpu.VMEM((2 * R_SC, LANES), jnp.int32),
            pltpu.VMEM((2 * CHUNK, width), jnp.float32),
            pltpu.VMEM((ZROWS, width), jnp.float32),
            pltpu.VMEM_SHARED((N_PAD, width), jnp.float32),
            pltpu.SemaphoreType.DMA,
            pltpu.SemaphoreType.DMA,
        ],
        compiler_params=pltpu.CompilerParams(use_tc_tiling_on_sc=False),
    )
    def scat(y_hbm, src_hbm, dst_hbm, out_hbm, sbuf, dbuf, rows, zbuf,
             agg_sh, sem0, sem1):
        cid = lax.axis_index("c")
        sid = lax.axis_index("s")
        gwid = sid * NC + cid
        nlanes = width // 16
        sems = (sem0, sem1)

        cnt = BASE_CNT + (gwid < REM).astype(jnp.int32)

        def fill(c, b):
            e0 = (gwid + NW * c) * CHUNK
            pltpu.sync_copy(src_hbm.at[pl.ds(e0, CHUNK)], sbuf.at[b])
            for j in range(R_SC):
                pltpu.sync_copy(dst_hbm.at[pl.ds(e0 + j * LANES, LANES)],
                                dbuf.at[b * R_SC + j])
            for j in range(R_SC):
                pltpu.async_copy(
                    y_hbm.at[sbuf.at[b, pl.ds(j * LANES, LANES)]],
                    rows.at[pl.ds((b * R_SC + j) * LANES, LANES)], sems[b])

        def drain(c, b):
            for j in range(R_SC):
                pltpu.make_async_copy(
                    y_hbm.at[sbuf.at[b, pl.ds(j * LANES, LANES)]],
                    rows.at[pl.ds((b * R_SC + j) * LANES, LANES)],
                    sems[b]).wait()
            for j in range(R_SC):
                pltpu.sync_copy(rows.at[pl.ds((b * R_SC + j) * LANES, LANES)],
                                agg_sh.at[dbuf.at[b * R_SC + j]], add=True)

        fill(0, 0)
        fill(1, 1)

        def zrow(i, carry):
            for l in range(nlanes):
                zbuf[i, pl.ds(l * 16, 16)] = jnp.zeros((16,), jnp.float32)
            return carry

        lax.fori_loop(0, ZROWS, zrow, 0)

        def zcopy(k, carry):
            pltpu.sync_copy(
                zbuf, agg_sh.at[pl.ds(sid * ROWS_PER_TILE + k * ZROWS, ZROWS)])
            return carry

        lax.fori_loop(0, ROWS_PER_TILE // ZROWS, zcopy, 0)
        plsc.subcore_barrier()

        def body(k, carry):
            for b in range(2):
                c = 2 * k + b

                @pl.when(c < cnt)
                def _():
                    drain(c, b)

                    @pl.when(c + 2 < cnt)
                    def _():
                        fill(c + 2, b)

            return carry

        lax.fori_loop(0, (MAX_CNT + 1) // 2, body, 0)
        plsc.subcore_barrier()
        pltpu.sync_copy(
            agg_sh.at[pl.ds(sid * ROWS_PER_TILE, ROWS_PER_TILE)],
            out_hbm.at[cid, pl.ds(sid * ROWS_PER_TILE, ROWS_PER_TILE)])

    return scat


_scatter_w1 = _make_scatter(W1)
_scatter_h = _make_scatter(HIDDEN)


def kernel(feats, edge_index, W_self1, W_neigh1, b1, W_self2, W_neigh2, b2,
           W_out, b_out):
    src = edge_index[0].astype(jnp.int32)
    dst = edge_index[1].astype(jnp.int32)
    wcat1 = jnp.concatenate([W_neigh1, W_self1], axis=1)

    y1, z1 = _stage1(feats, wcat1)
    agg1 = _scatter_w1(y1, src, dst)
    y2, z2, deg = _stage2(z1, agg1, b1.reshape(1, HIDDEN), W_neigh2, W_self2)
    agg2 = _scatter_h(y2, src, dst)

    wout_p = jnp.pad(W_out, ((0, 0), (0, 8 - W_out.shape[1])))
    bout_p = jnp.pad(b_out, (0, 8 - b_out.shape[0])).reshape(1, 8)
    out = _stage3(z2, agg2, deg, b2.reshape(1, HIDDEN), wout_p, bout_p)
    return out[:N_NODES, :W_out.shape[1]]


# speedup vs baseline: 1.2693x; 1.1042x over previous
import functools

import jax
import jax.numpy as jnp
from jax import lax
from jax.experimental import pallas as pl
from jax.experimental.pallas import tpu as pltpu
from jax.experimental.pallas import tpu_sc as plsc

N_NODES = 10000
N_PAD = 10240
N_EDGES = 320000
IN_FEATS = 128
HIDDEN = 64

NC = 2
NS = 16
NW = NC * NS
LANES = 128
R_SC = 2
CHUNK = R_SC * LANES
N_CHUNKS = N_EDGES // CHUNK
BASE_CNT = N_CHUNKS // NW
REM = N_CHUNKS % NW
MAX_CNT = BASE_CNT + 1
ROWS_PER_TILE = N_PAD // NS
ZROWS = 40

ROW_BLK = 1024
W1 = HIDDEN + 16

_P = jax.lax.Precision.HIGHEST


def _dot(a, b):
    return jax.lax.dot_general(a, b, (((1,), (0,)), ((), ())),
                               preferred_element_type=jnp.float32,
                               precision=_P)


def _stage1_body(feats_ref, wcat_ref, y_ref, z_ref):
    acc = _dot(feats_ref[...], wcat_ref[...])
    ones = jnp.ones((ROW_BLK, W1 - HIDDEN), jnp.float32)
    y_ref[...] = jnp.concatenate([acc[:, :HIDDEN], ones], axis=1)
    z_ref[...] = acc[:, HIDDEN:]


def _stage1(feats, wcat):
    return pl.pallas_call(
        _stage1_body,
        grid=(N_PAD // ROW_BLK,),
        in_specs=[
            pl.BlockSpec((ROW_BLK, IN_FEATS), lambda i: (i, 0)),
            pl.BlockSpec((IN_FEATS, 2 * HIDDEN), lambda i: (0, 0)),
        ],
        out_specs=[
            pl.BlockSpec((ROW_BLK, W1), lambda i: (i, 0)),
            pl.BlockSpec((ROW_BLK, HIDDEN), lambda i: (i, 0)),
        ],
        out_shape=[
            jax.ShapeDtypeStruct((N_PAD, W1), jnp.float32),
            jax.ShapeDtypeStruct((N_PAD, HIDDEN), jnp.float32),
        ],
    )(feats, wcat)


def _stage2_body(z1_ref, agg_ref, b1_ref, wn2_ref, ws2_ref,
                 y2_ref, z2_ref, deg_ref):
    a = agg_ref[0] + agg_ref[1]
    deg = jnp.maximum(a[:, HIDDEN:HIDDEN + 1], 1.0)
    mean = a[:, :HIDDEN] / deg
    h1 = jnp.maximum(z1_ref[...] + mean + b1_ref[...], 0.0)
    y2_ref[...] = _dot(h1, wn2_ref[...])
    z2_ref[...] = _dot(h1, ws2_ref[...])
    deg_ref[...] = jnp.broadcast_to(deg, (ROW_BLK, 8))


def _stage2(z1, agg1, b1, wn2, ws2):
    return pl.pallas_call(
        _stage2_body,
        grid=(N_PAD // ROW_BLK,),
        in_specs=[
            pl.BlockSpec((ROW_BLK, HIDDEN), lambda i: (i, 0)),
            pl.BlockSpec((NC, ROW_BLK, W1), lambda i: (0, i, 0)),
            pl.BlockSpec((1, HIDDEN), lambda i: (0, 0)),
            pl.BlockSpec((HIDDEN, HIDDEN), lambda i: (0, 0)),
            pl.BlockSpec((HIDDEN, HIDDEN), lambda i: (0, 0)),
        ],
        out_specs=[
            pl.BlockSpec((ROW_BLK, HIDDEN), lambda i: (i, 0)),
            pl.BlockSpec((ROW_BLK, HIDDEN), lambda i: (i, 0)),
            pl.BlockSpec((ROW_BLK, 8), lambda i: (i, 0)),
        ],
        out_shape=[
            jax.ShapeDtypeStruct((N_PAD, HIDDEN), jnp.float32),
            jax.ShapeDtypeStruct((N_PAD, HIDDEN), jnp.float32),
            jax.ShapeDtypeStruct((N_PAD, 8), jnp.float32),
        ],
    )(z1, agg1, b1, wn2, ws2)


def _stage3_body(z2_ref, agg_ref, deg_ref, b2_ref, wout_ref, bout_ref, o_ref):
    a = agg_ref[0] + agg_ref[1]
    mean = a / deg_ref[:, 0:1]
    h2 = jnp.maximum(z2_ref[...] + mean + b2_ref[...], 0.0)
    o_ref[...] = _dot(h2, wout_ref[...]) + bout_ref[...]


def _stage3(z2, agg2, deg, b2, wout_p, bout_p):
    return pl.pallas_call(
        _stage3_body,
        grid=(N_PAD // ROW_BLK,),
        in_specs=[
            pl.BlockSpec((ROW_BLK, HIDDEN), lambda i: (i, 0)),
            pl.BlockSpec((NC, ROW_BLK, HIDDEN), lambda i: (0, i, 0)),
            pl.BlockSpec((ROW_BLK, 8), lambda i: (i, 0)),
            pl.BlockSpec((1, HIDDEN), lambda i: (0, 0)),
            pl.BlockSpec((HIDDEN, 8), lambda i: (0, 0)),
            pl.BlockSpec((1, 8), lambda i: (0, 0)),
        ],
        out_specs=pl.BlockSpec((ROW_BLK, 8), lambda i: (i, 0)),
        out_shape=jax.ShapeDtypeStruct((N_PAD, 8), jnp.float32),
    )(z2, agg2, deg, b2, wout_p, bout_p)


def _make_scatter(width):
    mesh = plsc.VectorSubcoreMesh(core_axis_name="c", subcore_axis_name="s")

    @functools.partial(
        pl.kernel,
        out_type=jax.ShapeDtypeStruct((NC, N_PAD, width), jnp.float32),
        mesh=mesh,
        scratch_types=[
            pltpu.VMEM((2 * CHUNK,), jnp.int32),
            pltpu.VMEM((2, R_SC, LANES), jnp.int32),
            pltpu.VMEM((2 * CHUNK, width), jnp.float32),
            pltpu.VMEM((ZROWS, width), jnp.float32),
            pltpu.VMEM_SHARED((N_PAD, width), jnp.float32),
            pltpu.SemaphoreType.DMA,
            pltpu.SemaphoreType.DMA,
        ],
        compiler_params=pltpu.CompilerParams(use_tc_tiling_on_sc=False),
    )
    def scat(y_hbm, src_hbm, dst_hbm, out_hbm, sbuf, dbuf, rows, zbuf,
             agg_sh, sem0, sem1):
        cid = lax.axis_index("c")
        sid = lax.axis_index("s")
        gwid = sid * NC + cid
        nlanes = width // 16
        sems = (sem0, sem1)

        cnt = BASE_CNT + (gwid < REM).astype(jnp.int32)

        def fill(c, b):
            q = gwid + NW * c
            pltpu.sync_copy(src_hbm.at[q], sbuf.at[pl.ds(b * CHUNK, CHUNK)])
            pltpu.sync_copy(dst_hbm.at[q], dbuf.at[b])
            for j in range(R_SC):
                pltpu.async_copy(
                    y_hbm.at[sbuf.at[pl.ds(b * CHUNK + j * LANES, LANES)]],
                    rows.at[pl.ds((b * R_SC + j) * LANES, LANES)], sems[b])

        def drain(c, b):
            for j in range(R_SC):
                pltpu.make_async_copy(
                    y_hbm.at[sbuf.at[pl.ds(b * CHUNK + j * LANES, LANES)]],
                    rows.at[pl.ds((b * R_SC + j) * LANES, LANES)],
                    sems[b]).wait()
            for j in range(R_SC):
                pltpu.sync_copy(rows.at[pl.ds((b * R_SC + j) * LANES, LANES)],
                                agg_sh.at[dbuf.at[b].at[j]], add=True)

        fill(0, 0)
        fill(1, 1)

        def zrow(i, carry):
            for l in range(nlanes):
                zbuf[i, pl.ds(l * 16, 16)] = jnp.zeros((16,), jnp.float32)
            return carry

        lax.fori_loop(0, ZROWS, zrow, 0)

        def zcopy(k, carry):
            pltpu.sync_copy(
                zbuf, agg_sh.at[pl.ds(sid * ROWS_PER_TILE + k * ZROWS, ZROWS)])
            return carry

        lax.fori_loop(0, ROWS_PER_TILE // ZROWS, zcopy, 0)
        plsc.subcore_barrier()

        def body(k, carry):
            for b in range(2):
                c = 2 * k + b

                @pl.when(c < cnt)
                def _():
                    drain(c, b)

                    @pl.when(c + 2 < cnt)
                    def _():
                        fill(c + 2, b)

            return carry

        lax.fori_loop(0, (MAX_CNT + 1) // 2, body, 0)
        plsc.subcore_barrier()
        pltpu.sync_copy(
            agg_sh.at[pl.ds(sid * ROWS_PER_TILE, ROWS_PER_TILE)],
            out_hbm.at[cid, pl.ds(sid * ROWS_PER_TILE, ROWS_PER_TILE)])

    return scat


_scatter_w1 = _make_scatter(W1)
_scatter_h = _make_scatter(HIDDEN)


def kernel(feats, edge_index, W_self1, W_neigh1, b1, W_self2, W_neigh2, b2,
           W_out, b_out):
    src = edge_index[0].astype(jnp.int32).reshape(N_CHUNKS, CHUNK)
    dst = edge_index[1].astype(jnp.int32).reshape(N_CHUNKS, R_SC, LANES)
    wcat1 = jnp.concatenate([W_neigh1, W_self1], axis=1)

    y1, z1 = _stage1(feats, wcat1)
    agg1 = _scatter_w1(y1, src, dst)
    y2, z2, deg = _stage2(z1, agg1, b1.reshape(1, HIDDEN), W_neigh2, W_self2)
    agg2 = _scatter_h(y2, src, dst)

    wout_p = jnp.pad(W_out, ((0, 0), (0, 8 - W_out.shape[1])))
    bout_p = jnp.pad(b_out, (0, 8 - b_out.shape[0])).reshape(1, 8)
    out = _stage3(z2, agg2, deg, b2.reshape(1, HIDDEN), wout_p, bout_p)
    return out[:N_NODES, :W_out.shape[1]]


# speedup vs baseline: 1.3416x; 1.0570x over previous
import functools

import jax
import jax.numpy as jnp
from jax import lax
from jax.experimental import pallas as pl
from jax.experimental.pallas import tpu as pltpu
from jax.experimental.pallas import tpu_sc as plsc

N_NODES = 10000
N_PAD = 10240
N_EDGES = 320000
IN_FEATS = 128
HIDDEN = 64

NC = 2
NS = 16
NW = NC * NS
LANES = 128
R_SC = 2
CHUNK = R_SC * LANES
N_CHUNKS = N_EDGES // CHUNK
BASE_CNT = N_CHUNKS // NW
REM = N_CHUNKS % NW
MAX_CNT = BASE_CNT + 1
ROWS_PER_TILE = N_PAD // NS
ZROWS = 40

ROW_BLK = 1024
W1 = HIDDEN + 16

_P = jax.lax.Precision.HIGHEST


def _dot(a, b):
    return jax.lax.dot_general(a, b, (((1,), (0,)), ((), ())),
                               preferred_element_type=jnp.float32,
                               precision=_P)


def _stage1_body(feats_ref, wcat_ref, y_ref, z_ref):
    acc = _dot(feats_ref[...], wcat_ref[...])
    ones = jnp.ones((ROW_BLK, W1 - HIDDEN), jnp.float32)
    y_ref[...] = jnp.concatenate([acc[:, :HIDDEN], ones], axis=1)
    z_ref[...] = acc[:, HIDDEN:]


def _stage1(feats, wcat):
    return pl.pallas_call(
        _stage1_body,
        grid=(N_PAD // ROW_BLK,),
        in_specs=[
            pl.BlockSpec((ROW_BLK, IN_FEATS), lambda i: (i, 0)),
            pl.BlockSpec((IN_FEATS, 2 * HIDDEN), lambda i: (0, 0)),
        ],
        out_specs=[
            pl.BlockSpec((ROW_BLK, W1), lambda i: (i, 0)),
            pl.BlockSpec((ROW_BLK, HIDDEN), lambda i: (i, 0)),
        ],
        out_shape=[
            jax.ShapeDtypeStruct((N_PAD, W1), jnp.float32),
            jax.ShapeDtypeStruct((N_PAD, HIDDEN), jnp.float32),
        ],
    )(feats, wcat)


def _stage2_body(z1_ref, agg_ref, b1_ref, wn2_ref, ws2_ref,
                 y2_ref, z2_ref, deg_ref):
    a = agg_ref[0] + agg_ref[1]
    deg = jnp.maximum(a[:, HIDDEN:HIDDEN + 1], 1.0)
    mean = a[:, :HIDDEN] / deg
    h1 = jnp.maximum(z1_ref[...] + mean + b1_ref[...], 0.0)
    y2_ref[...] = _dot(h1, wn2_ref[...])
    z2_ref[...] = _dot(h1, ws2_ref[...])
    deg_ref[...] = jnp.broadcast_to(deg, (ROW_BLK, 8))


def _stage2(z1, agg1, b1, wn2, ws2):
    return pl.pallas_call(
        _stage2_body,
        grid=(N_PAD // ROW_BLK,),
        in_specs=[
            pl.BlockSpec((ROW_BLK, HIDDEN), lambda i: (i, 0)),
            pl.BlockSpec((NC, ROW_BLK, W1), lambda i: (0, i, 0)),
            pl.BlockSpec((1, HIDDEN), lambda i: (0, 0)),
            pl.BlockSpec((HIDDEN, HIDDEN), lambda i: (0, 0)),
            pl.BlockSpec((HIDDEN, HIDDEN), lambda i: (0, 0)),
        ],
        out_specs=[
            pl.BlockSpec((ROW_BLK, HIDDEN), lambda i: (i, 0)),
            pl.BlockSpec((ROW_BLK, HIDDEN), lambda i: (i, 0)),
            pl.BlockSpec((ROW_BLK, 8), lambda i: (i, 0)),
        ],
        out_shape=[
            jax.ShapeDtypeStruct((N_PAD, HIDDEN), jnp.float32),
            jax.ShapeDtypeStruct((N_PAD, HIDDEN), jnp.float32),
            jax.ShapeDtypeStruct((N_PAD, 8), jnp.float32),
        ],
    )(z1, agg1, b1, wn2, ws2)


def _stage3_body(z2_ref, agg_ref, deg_ref, b2_ref, wout_ref, bout_ref, o_ref):
    a = agg_ref[0] + agg_ref[1]
    mean = a / deg_ref[:, 0:1]
    h2 = jnp.maximum(z2_ref[...] + mean + b2_ref[...], 0.0)
    o_ref[...] = _dot(h2, wout_ref[...]) + bout_ref[...]


def _stage3(z2, agg2, deg, b2, wout_p, bout_p):
    return pl.pallas_call(
        _stage3_body,
        grid=(N_PAD // ROW_BLK,),
        in_specs=[
            pl.BlockSpec((ROW_BLK, HIDDEN), lambda i: (i, 0)),
            pl.BlockSpec((NC, ROW_BLK, HIDDEN), lambda i: (0, i, 0)),
            pl.BlockSpec((ROW_BLK, 8), lambda i: (i, 0)),
            pl.BlockSpec((1, HIDDEN), lambda i: (0, 0)),
            pl.BlockSpec((HIDDEN, 8), lambda i: (0, 0)),
            pl.BlockSpec((1, 8), lambda i: (0, 0)),
        ],
        out_specs=pl.BlockSpec((ROW_BLK, 8), lambda i: (i, 0)),
        out_shape=jax.ShapeDtypeStruct((N_PAD, 8), jnp.float32),
    )(z2, agg2, deg, b2, wout_p, bout_p)


def _make_scatter(width, r_sc):
    mesh = plsc.VectorSubcoreMesh(core_axis_name="c", subcore_axis_name="s")
    chunk = r_sc * LANES
    n_chunks = N_EDGES // chunk
    base_cnt = n_chunks // NW
    rem = n_chunks % NW
    max_cnt = base_cnt + (1 if rem else 0)

    @functools.partial(
        pl.kernel,
        out_type=jax.ShapeDtypeStruct((NC, N_PAD, width), jnp.float32),
        mesh=mesh,
        scratch_types=[
            pltpu.VMEM((2 * chunk,), jnp.int32),
            pltpu.VMEM((2, r_sc, LANES), jnp.int32),
            pltpu.VMEM((2 * chunk, width), jnp.float32),
            pltpu.VMEM((ZROWS, width), jnp.float32),
            pltpu.VMEM_SHARED((N_PAD, width), jnp.float32),
            pltpu.SemaphoreType.DMA,
            pltpu.SemaphoreType.DMA,
        ],
        compiler_params=pltpu.CompilerParams(use_tc_tiling_on_sc=False),
    )
    def scat(y_hbm, src_hbm, dst_hbm, out_hbm, sbuf, dbuf, rows, zbuf,
             agg_sh, sem0, sem1):
        cid = lax.axis_index("c")
        sid = lax.axis_index("s")
        gwid = sid * NC + cid
        nlanes = width // 16
        sems = (sem0, sem1)

        cnt = base_cnt + (gwid < rem).astype(jnp.int32)

        def fill(c, b):
            q = gwid + NW * c
            pltpu.sync_copy(src_hbm.at[q], sbuf.at[pl.ds(b * chunk, chunk)])
            pltpu.sync_copy(dst_hbm.at[q], dbuf.at[b])
            for j in range(r_sc):
                pltpu.async_copy(
                    y_hbm.at[sbuf.at[pl.ds(b * chunk + j * LANES, LANES)]],
                    rows.at[pl.ds((b * r_sc + j) * LANES, LANES)], sems[b])

        def drain(c, b):
            for j in range(r_sc):
                pltpu.make_async_copy(
                    y_hbm.at[sbuf.at[pl.ds(b * chunk + j * LANES, LANES)]],
                    rows.at[pl.ds((b * r_sc + j) * LANES, LANES)],
                    sems[b]).wait()
            for j in range(r_sc):
                pltpu.sync_copy(rows.at[pl.ds((b * r_sc + j) * LANES, LANES)],
                                agg_sh.at[dbuf.at[b].at[j]], add=True)

        fill(0, 0)
        fill(1, 1)

        def zrow(i, carry):
            for l in range(nlanes):
                zbuf[i, pl.ds(l * 16, 16)] = jnp.zeros((16,), jnp.float32)
            return carry

        lax.fori_loop(0, ZROWS, zrow, 0)

        def zcopy(k, carry):
            pltpu.sync_copy(
                zbuf, agg_sh.at[pl.ds(sid * ROWS_PER_TILE + k * ZROWS, ZROWS)])
            return carry

        lax.fori_loop(0, ROWS_PER_TILE // ZROWS, zcopy, 0)
        plsc.subcore_barrier()

        def body(k, carry):
            for b in range(2):
                c = 2 * k + b

                @pl.when(c < cnt)
                def _():
                    drain(c, b)

                    @pl.when(c + 2 < cnt)
                    def _():
                        fill(c + 2, b)

            return carry

        lax.fori_loop(0, (max_cnt + 1) // 2, body, 0)
        plsc.subcore_barrier()
        pltpu.sync_copy(
            agg_sh.at[pl.ds(sid * ROWS_PER_TILE, ROWS_PER_TILE)],
            out_hbm.at[cid, pl.ds(sid * ROWS_PER_TILE, ROWS_PER_TILE)])

    return scat


_scatter_w1 = _make_scatter(W1, R_SC)
_scatter_h = _make_scatter(HIDDEN, 2 * R_SC)


def kernel(feats, edge_index, W_self1, W_neigh1, b1, W_self2, W_neigh2, b2,
           W_out, b_out):
    srcf = edge_index[0].astype(jnp.int32)
    dstf = edge_index[1].astype(jnp.int32)
    src = srcf.reshape(N_CHUNKS, CHUNK)
    dst = dstf.reshape(N_CHUNKS, R_SC, LANES)
    src2 = srcf.reshape(N_CHUNKS // 2, 2 * CHUNK)
    dst2 = dstf.reshape(N_CHUNKS // 2, 2 * R_SC, LANES)
    wcat1 = jnp.concatenate([W_neigh1, W_self1], axis=1)

    y1, z1 = _stage1(feats, wcat1)
    agg1 = _scatter_w1(y1, src, dst)
    y2, z2, deg = _stage2(z1, agg1, b1.reshape(1, HIDDEN), W_neigh2, W_self2)
    agg2 = _scatter_h(y2, src2, dst2)

    wout_p = jnp.pad(W_out, ((0, 0), (0, 8 - W_out.shape[1])))
    bout_p = jnp.pad(b_out, (0, 8 - b_out.shape[0])).reshape(1, 8)
    out = _stage3(z2, agg2, deg, b2.reshape(1, HIDDEN), wout_p, bout_p)
    return out[:N_NODES, :W_out.shape[1]]
